# Initial kernel scaffold; baseline (speedup 1.0000x reference)
import jax
import jax.numpy as jnp
from jax.experimental import pallas as pl


def kernel(x_src, x_ref, src_edge_indices, ref_edge_indices, W_l, b_l, W_r):
    raise NotImplementedError("write your pallas kernel here")


# speedup vs baseline: 3.1864x; 3.1864x over previous
import functools

import jax
import jax.numpy as jnp
from jax import lax
from jax.experimental import pallas as pl
from jax.experimental.pallas import tpu as pltpu
from jax.experimental.pallas import tpu_sc as plsc

NC = 2
NS = 16
NW = NC * NS
L = 16
CHUNK = 128


@functools.lru_cache(maxsize=None)
def _make_sc_segsum(N, D, CH, NPAD):
    ROWS_PT = NPAD // NS
    mesh = plsc.VectorSubcoreMesh(core_axis_name="c", subcore_axis_name="s")

    @functools.partial(
        pl.kernel,
        out_type=(
            jax.ShapeDtypeStruct((NC, NPAD, D), jnp.float32),
            jax.ShapeDtypeStruct((NC, NS, NPAD), jnp.float32),
        ),
        mesh=mesh,
        compiler_params=pltpu.CompilerParams(needs_layout_passes=False),
        scratch_types=(
            pltpu.VMEM_SHARED((NPAD, D), jnp.float32),
            pltpu.VMEM((CH, CHUNK), jnp.int32),
            pltpu.VMEM((CH, CHUNK), jnp.int32),
            pltpu.VMEM((CHUNK, D), jnp.float32),
            pltpu.VMEM((NPAD,), jnp.float32),
        ),
    )
    def seg(x_hbm, srcs_hbm, dsts_hbm, zeros_hbm, agg_out, deg_out,
            shared_agg, src_v, dst_v, rows_v, deg_v):
        c = lax.axis_index("c")
        s = lax.axis_index("s")
        wid = c * NS + s
        pltpu.sync_copy(zeros_hbm.at[pl.ds(s * ROWS_PT, ROWS_PT)],
                        shared_agg.at[pl.ds(s * ROWS_PT, ROWS_PT)])
        zeros16 = jnp.zeros((L,), jnp.float32)

        def zero_body(i, carry):
            deg_v[pl.ds(i * L, L)] = zeros16
            return carry

        lax.fori_loop(0, NPAD // L, zero_body, 0)
        pltpu.sync_copy(srcs_hbm.at[wid], src_v)
        pltpu.sync_copy(dsts_hbm.at[wid], dst_v)
        plsc.subcore_barrier()

        ones = jnp.full((L,), 1.0, jnp.float32)

        def chunk_body(j, carry):
            pltpu.sync_copy(x_hbm.at[src_v.at[j]], rows_v)
            pltpu.sync_copy(rows_v, shared_agg.at[dst_v.at[j]], add=True)
            for k in range(CHUNK // L):
                d = dst_v[j, pl.ds(k * L, L)]
                plsc.addupdate_scatter(deg_v, [d], ones)
            return carry

        lax.fori_loop(0, CH, chunk_body, 0)
        plsc.subcore_barrier()
        pltpu.sync_copy(shared_agg.at[pl.ds(s * ROWS_PT, ROWS_PT)],
                        agg_out.at[c, pl.ds(s * ROWS_PT, ROWS_PT)])
        pltpu.sync_copy(deg_v, deg_out.at[c, s])

    return seg


def _finalize_body(x_ref, agg_ref, deg_ref, wl_ref, bl_ref, wr_ref, o_ref):
    agg = agg_ref[0, 0] + agg_ref[0, 1]
    deg = jnp.sum(deg_ref[0], axis=0)
    mean = agg / jnp.clip(deg, 1.0)[:, None]
    dn = (((1,), (1,)), ((), ()))
    o_ref[0] = (
        lax.dot_general(mean, wl_ref[...], dn, preferred_element_type=jnp.float32)
        + lax.dot_general(x_ref[0], wr_ref[...], dn, preferred_element_type=jnp.float32)
        + bl_ref[...])


def kernel(x_src, x_ref, src_edge_indices, ref_edge_indices, W_l, b_l, W_r):
    N, D = x_src.shape
    E = src_edge_indices.shape[0]
    CH = -(-E // (NW * CHUNK))
    CH += CH % 2
    EP = NW * CH * CHUNK
    NPAD = -(-(N + 1) // 2048) * 2048

    def prep(edges):
        pad = EP - E
        src = jnp.concatenate(
            [edges[:, 0], jnp.zeros((pad,), jnp.int32)]).reshape(NW, CH, CHUNK)
        dst = jnp.concatenate(
            [edges[:, 1], jnp.full((pad,), N, jnp.int32)]).reshape(NW, CH, CHUNK)
        return src, dst

    ss, ds = prep(src_edge_indices)
    sr, dr = prep(ref_edge_indices)
    zeros = jnp.zeros((NPAD, D), jnp.float32)

    seg = _make_sc_segsum(N, D, CH, NPAD)
    agg_s, deg_s = seg(x_src, ss, ds, zeros)
    agg_r, deg_r = seg(x_ref, sr, dr, zeros)

    xs = jnp.stack([x_src, x_ref])
    aggs = jnp.stack([agg_s, agg_r])
    degs = jnp.stack(
        [deg_s.reshape(NW, NPAD), deg_r.reshape(NW, NPAD)])

    RB = 512
    nb = NPAD // RB
    outs = pl.pallas_call(
        _finalize_body,
        grid=(2, nb),
        in_specs=[
            pl.BlockSpec((1, RB, D), lambda g, i: (g, i, 0)),
            pl.BlockSpec((1, NC, RB, D), lambda g, i: (g, 0, i, 0)),
            pl.BlockSpec((1, NW, RB), lambda g, i: (g, 0, i)),
            pl.BlockSpec((D, D), lambda g, i: (0, 0)),
            pl.BlockSpec((1, D), lambda g, i: (0, 0)),
            pl.BlockSpec((D, D), lambda g, i: (0, 0)),
        ],
        out_specs=pl.BlockSpec((1, RB, D), lambda g, i: (g, i, 0)),
        out_shape=jax.ShapeDtypeStruct((2, N, D), jnp.float32),
    )(xs, aggs, degs, W_l, b_l.reshape(1, D), W_r)
    return outs[0], outs[1]


# speedup vs baseline: 3.7644x; 1.1814x over previous
import functools

import jax
import jax.numpy as jnp
from jax import lax
from jax.experimental import pallas as pl
from jax.experimental.pallas import tpu as pltpu
from jax.experimental.pallas import tpu_sc as plsc

NC = 2
NS = 16
NW = NC * NS
L = 16
CHUNK = 64
NBUF = 2
IG = 32


@functools.lru_cache(maxsize=None)
def _make_sc_segsum(N, D, CH, NPAD):
    ROWS_PT = NPAD // NS
    mesh = plsc.VectorSubcoreMesh(core_axis_name="c", subcore_axis_name="s")

    @functools.partial(
        pl.kernel,
        out_type=(
            jax.ShapeDtypeStruct((NC, NPAD, D), jnp.float32),
            jax.ShapeDtypeStruct((NC, NS, NPAD), jnp.float32),
        ),
        mesh=mesh,
        compiler_params=pltpu.CompilerParams(needs_layout_passes=False),
        scratch_types=(
            pltpu.VMEM_SHARED((NPAD, D), jnp.float32),
            pltpu.VMEM((2, IG, CHUNK), jnp.int32),
            pltpu.VMEM((2, IG, CHUNK), jnp.int32),
            pltpu.VMEM((NBUF, CHUNK, D), jnp.float32),
            pltpu.VMEM((NPAD,), jnp.float32),
            pltpu.SemaphoreType.DMA,
            pltpu.SemaphoreType.DMA,
            pltpu.SemaphoreType.DMA,
        ),
    )
    def seg(x_hbm, srcs_hbm, dsts_hbm, zeros_hbm, agg_out, deg_out,
            shared_agg, src_v, dst_v, rows_v, deg_v, s0, s1, si):
        sems = (s0, s1)
        NG = CH // IG
        c = lax.axis_index("c")
        s = lax.axis_index("s")
        wid = c * NS + s
        pltpu.sync_copy(zeros_hbm.at[pl.ds(s * ROWS_PT, ROWS_PT)],
                        shared_agg.at[pl.ds(s * ROWS_PT, ROWS_PT)])
        zeros16 = jnp.zeros((L,), jnp.float32)

        def zero_body(i, carry):
            deg_v[pl.ds(i * L, L)] = zeros16
            return carry

        lax.fori_loop(0, NPAD // L, zero_body, 0)
        pltpu.sync_copy(srcs_hbm.at[wid, pl.ds(0, IG)], src_v.at[0])
        pltpu.sync_copy(dsts_hbm.at[wid, pl.ds(0, IG)], dst_v.at[0])
        plsc.subcore_barrier()

        ones = jnp.full((L,), 1.0, jnp.float32)
        for b in range(NBUF):
            pltpu.async_copy(x_hbm.at[src_v.at[0, b]], rows_v.at[b], sems[b])

        def group_body(g, carry):
            slot = lax.rem(g, 2)
            nslot = lax.rem(g + 1, 2)

            @pl.when(g + 1 < NG)
            def _():
                pltpu.async_copy(
                    srcs_hbm.at[wid, pl.ds((g + 1) * IG, IG)],
                    src_v.at[nslot], si)
                pltpu.async_copy(
                    dsts_hbm.at[wid, pl.ds((g + 1) * IG, IG)],
                    dst_v.at[nslot], si)

            for jj in range(IG):
                j = g * IG + jj
                b = jj % NBUF
                for k in range(CHUNK // L):
                    d = dst_v[slot, jj, pl.ds(k * L, L)]
                    plsc.addupdate_scatter(deg_v, [d], ones)
                if jj == IG - NBUF:
                    @pl.when(g + 1 < NG)
                    def _():
                        pltpu.make_async_copy(
                            srcs_hbm.at[wid, pl.ds((g + 1) * IG, IG)],
                            src_v.at[nslot], si).wait()
                        pltpu.make_async_copy(
                            dsts_hbm.at[wid, pl.ds((g + 1) * IG, IG)],
                            dst_v.at[nslot], si).wait()
                pltpu.make_async_copy(
                    x_hbm.at[src_v.at[slot, jj]], rows_v.at[b], sems[b]).wait()
                pltpu.sync_copy(rows_v.at[b], shared_agg.at[dst_v.at[slot, jj]],
                                add=True)

                @pl.when(j + NBUF < CH)
                def _():
                    if jj + NBUF < IG:
                        pltpu.async_copy(
                            x_hbm.at[src_v.at[slot, jj + NBUF]],
                            rows_v.at[b], sems[b])
                    else:
                        pltpu.async_copy(
                            x_hbm.at[src_v.at[nslot, jj + NBUF - IG]],
                            rows_v.at[b], sems[b])
            return carry

        lax.fori_loop(0, NG, group_body, 0)
        plsc.subcore_barrier()
        pltpu.sync_copy(shared_agg.at[pl.ds(s * ROWS_PT, ROWS_PT)],
                        agg_out.at[c, pl.ds(s * ROWS_PT, ROWS_PT)])
        pltpu.sync_copy(deg_v, deg_out.at[c, s])

    return seg


def _finalize_body(x_ref, agg_ref, deg_ref, wl_ref, bl_ref, wr_ref, o_ref):
    agg = agg_ref[0] + agg_ref[1]
    deg = jnp.sum(deg_ref[...], axis=0)
    mean = agg / jnp.clip(deg, 1.0)[:, None]
    dn = (((1,), (1,)), ((), ()))
    o_ref[...] = (
        lax.dot_general(mean, wl_ref[...], dn, preferred_element_type=jnp.float32)
        + lax.dot_general(x_ref[...], wr_ref[...], dn, preferred_element_type=jnp.float32)
        + bl_ref[...])


def kernel(x_src, x_ref, src_edge_indices, ref_edge_indices, W_l, b_l, W_r):
    N, D = x_src.shape
    E = src_edge_indices.shape[0]
    CH = -(-E // (NW * CHUNK))
    CH = -(-CH // IG) * IG
    EP = NW * CH * CHUNK
    NPAD = -(-(N + 1) // 128) * 128

    def prep(edges):
        pad = EP - E
        src = jnp.concatenate(
            [edges[:, 0], jnp.zeros((pad,), jnp.int32)]).reshape(NW, CH, CHUNK)
        dst = jnp.concatenate(
            [edges[:, 1], jnp.full((pad,), N, jnp.int32)]).reshape(NW, CH, CHUNK)
        return src, dst

    ss, ds = prep(src_edge_indices)
    sr, dr = prep(ref_edge_indices)
    zeros = jnp.zeros((NPAD, D), jnp.float32)

    seg = _make_sc_segsum(N, D, CH, NPAD)
    agg_s, deg_s = seg(x_src, ss, ds, zeros)
    agg_r, deg_r = seg(x_ref, sr, dr, zeros)

    RB = 512
    nb = -(-N // RB)
    fin = pl.pallas_call(
        _finalize_body,
        grid=(nb,),
        in_specs=[
            pl.BlockSpec((RB, D), lambda i: (i, 0)),
            pl.BlockSpec((NC, RB, D), lambda i: (0, i, 0)),
            pl.BlockSpec((NW, RB), lambda i: (0, i)),
            pl.BlockSpec((D, D), lambda i: (0, 0)),
            pl.BlockSpec((1, D), lambda i: (0, 0)),
            pl.BlockSpec((D, D), lambda i: (0, 0)),
        ],
        out_specs=pl.BlockSpec((RB, D), lambda i: (i, 0)),
        out_shape=jax.ShapeDtypeStruct((N, D), jnp.float32),
    )
    bl2 = b_l.reshape(1, D)
    out_src = fin(x_src, agg_s, deg_s.reshape(NW, NPAD), W_l, bl2, W_r)
    out_ref = fin(x_ref, agg_r, deg_r.reshape(NW, NPAD), W_l, bl2, W_r)
    return out_src, out_ref
